# baseline (speedup 1.0000x reference)
import jax
import jax.numpy as jnp
from jax.experimental import pallas as pl
from jax.experimental.pallas import tpu as pltpu

GAMMA = 1.0
LBD_ORD = 1.0
LBD_DUR = 5.0
LBD_LEN = 0.2
SIGMA = 1.0
MARGIN = 1.0

SUB = 8
LANES = 640
BLK = 128
NBLK = LANES // BLK


def _dp_kernel(ns_ref, scal_ref, dpad_ref, bld_ref, kps_ref, kus_ref, out_ref):
    lanes = jax.lax.broadcasted_iota(jnp.int32, (SUB, LANES), 1).astype(jnp.float32)
    scal = scal_ref[...]
    Mf = scal[:, 0:1]
    Nf = scal[:, 1:2]
    wsf = scal[:, 2:3]
    lds = scal[:, 3:4]
    ldo = scal[:, 4:5]

    one0 = jnp.where(lanes == 0.0, 1.0, 0.0)
    p0 = one0
    u0 = one0
    t0 = jnp.zeros((SUB, LANES), jnp.float32)
    z1 = jnp.zeros((SUB, 1), jnp.float32)
    o1 = jnp.ones((SUB, 1), jnp.float32)

    def body(i, carry):
        pc, uc, tacc, lkp, lku, spm, sum_ = carry
        fi = i.astype(jnp.float32)
        epb = pc.astype(jnp.bfloat16)
        eub = uc.astype(jnp.bfloat16)

        zblk = jnp.zeros((SUB, BLK), jnp.bfloat16)
        ep_blocks = [epb[:, b * BLK : (b + 1) * BLK] for b in range(NBLK)] + [zblk]
        eu_blocks = [zblk] + [eub[:, b * BLK : (b + 1) * BLK] for b in range(NBLK)]
        tp = [jnp.sum(pc[:, b * BLK : (b + 1) * BLK], axis=1, keepdims=True)
              for b in range(NBLK)]
        tu = [jnp.sum(uc[:, b * BLK : (b + 1) * BLK], axis=1, keepdims=True)
              for b in range(NBLK)]

        p_in = jnp.concatenate(
            [jnp.concatenate([ep_blocks[b], ep_blocks[b + 1]], axis=1)
             for b in range(NBLK)], axis=0)
        u_in = jnp.concatenate(
            [jnp.concatenate([eu_blocks[b], eu_blocks[b + 1]], axis=1)
             for b in range(NBLK)], axis=0)
        sp_all = jnp.dot(p_in, kps_ref[...], preferred_element_type=jnp.float32)
        su_all = jnp.dot(u_in, kus_ref[...], preferred_element_type=jnp.float32)

        einv = jnp.float32(0.36787944117144233)
        sp_blocks = []
        su_blocks = []
        cpre = z1
        csuf = z1
        for b in range(NBLK):
            sp_blocks.append(sp_all[b * SUB : (b + 1) * SUB] + cpre)
            cpre = cpre + tp[b]
        for b in range(NBLK - 1, -1, -1):
            su_blocks.append(su_all[b * SUB : (b + 1) * SUB] + einv * csuf)
            csuf = csuf + tu[b]
        sp = jnp.concatenate(sp_blocks, axis=1)
        su = jnp.concatenate(su_blocks[::-1], axis=1)

        dvalrow = dpad_ref[i] + bld_ref[i] * lds + ldo
        lo = jnp.maximum(1.0, fi - wsf)
        hi = jnp.minimum(Nf + 1.0, fi + wsf)
        bandt = (lanes >= lo) & (lanes < hi)
        dexp = jnp.where(bandt, jnp.exp(-dvalrow), 0.0)
        dexp_p = dexp * (1.0 / spm)
        dexp_u = dexp * (1.0 / sum_)

        tval = dvalrow + jnp.minimum(
            lkp - jnp.log(sp), lku - lanes - jnp.log(su)
        )
        hit = (fi == Mf) & (lanes == Nf)
        tacc = tacc + jnp.where(hit, tval, 0.0)

        lkp2 = lkp - jnp.log(spm)
        lku2 = lku - jnp.log(sum_)
        pn = dexp_p * sp
        un = dexp_u * su
        spm2 = jnp.max(pn, axis=1, keepdims=True)
        sum2 = jnp.max(un, axis=1, keepdims=True)
        return pn, un, tacc, lkp2, lku2, spm2, sum2

    def body8(idx, carry):
        for r in range(16):
            carry = body(16 * idx + 1 + r, carry)
        return carry

    carry = (p0, u0, t0, z1, z1, o1, o1)
    carry = jax.lax.fori_loop(0, ns_ref[0], body8, carry)
    tacc = carry[2]
    out_ref[...] = jnp.broadcast_to(
        jnp.sum(tacc, axis=1, keepdims=True), (SUB, 128)
    )


@jax.jit
def kernel(D, len_a, len_b):
    B, L1, L2 = D.shape

    Mf = len_a.astype(jnp.float32)
    Nf = len_b.astype(jnp.float32)
    dM = jnp.abs(Mf - Nf)
    wsf = jnp.maximum(5.0, dM + 1.0)
    lds = LBD_DUR / Mf
    ldo = dM * LBD_LEN / Mf
    cols = jnp.stack([Mf, Nf, wsf, lds, ldo], axis=1)
    scal = jnp.zeros((SUB, 128), jnp.float32)
    scal = scal.at[:B, :5].set(cols)
    scal = scal.at[B:, 0].set(256.0)
    scal = scal.at[B:, 1].set(256.0)
    scal = scal.at[B:, 2].set(5.0)
    scal = scal.at[B:, 3].set(LBD_DUR / 256.0)

    nsteps = (jnp.max(len_a).astype(jnp.int32) + 15) // 16

    Dr = jnp.concatenate([D, D[:, -1:, :]], axis=1)
    Dt = jnp.transpose(Dr, (1, 0, 2))
    dpad = jnp.pad(Dt, ((1, 0), (0, SUB - B), (1, LANES - L2 - 1)))

    ii = jnp.arange(L1 + 2, dtype=jnp.float32)[:, None]
    jj = jnp.arange(LANES, dtype=jnp.float32)[None, :]
    bld = 1.0 - jnp.exp(-((ii - jj) ** 2) / (2.0 * SIGMA**2 * (jj + 1.0)))
    bld = bld.reshape(L1 + 2, 1, LANES)

    t = jnp.arange(BLK, dtype=jnp.float32)[:, None]
    s = jnp.arange(BLK, dtype=jnp.float32)[None, :]
    kp_in = jnp.where(t <= s - 1.0, 1.0, jnp.exp(-(t - s + 1.0)))
    kp_nx = jnp.exp(-(BLK + t - s + 1.0))
    ku_in = jnp.where(t <= s - 1.0, jnp.exp(-(s - t)), jnp.exp(-1.0))
    ku_pv = jnp.exp(-(s + BLK - t))
    kps = jnp.concatenate([kp_in, kp_nx], axis=0).astype(jnp.bfloat16)
    kus = jnp.concatenate([ku_pv, ku_in], axis=0).astype(jnp.bfloat16)

    out = pl.pallas_call(
        _dp_kernel,
        out_shape=jax.ShapeDtypeStruct((SUB, 128), jnp.float32),
        in_specs=[
            pl.BlockSpec(memory_space=pltpu.SMEM),
            pl.BlockSpec(memory_space=pltpu.VMEM),
            pl.BlockSpec(memory_space=pltpu.VMEM),
            pl.BlockSpec(memory_space=pltpu.VMEM),
            pl.BlockSpec(memory_space=pltpu.VMEM),
            pl.BlockSpec(memory_space=pltpu.VMEM),
        ],
    )(nsteps.reshape(1), scal, dpad, bld, kps, kus)
    return out[:B, 0]


# speedup vs baseline: 1.0490x; 1.0490x over previous
import jax
import jax.numpy as jnp
from jax.experimental import pallas as pl
from jax.experimental.pallas import tpu as pltpu

GAMMA = 1.0
LBD_ORD = 1.0
LBD_DUR = 5.0
LBD_LEN = 0.2
SIGMA = 1.0
MARGIN = 1.0

import numpy as np

SUB = 8
LANES = 640
BLK = 128
NBLK = LANES // BLK
NROWS = 514


def _build_tables():
    ii = np.arange(NROWS, dtype=np.float64)[:, None]
    jj = np.arange(LANES, dtype=np.float64)[None, :]
    bld = (1.0 - np.exp(-((ii - jj) ** 2) / (2.0 * SIGMA**2 * (jj + 1.0))))
    bld = bld.astype(np.float32).reshape(NROWS, 1, LANES)
    t = np.arange(BLK, dtype=np.float64)[:, None]
    s = np.arange(BLK, dtype=np.float64)[None, :]
    kp_in = np.where(t <= s - 1.0, 1.0, np.exp(-(t - s + 1.0)))
    kp_nx = np.exp(-(BLK + t - s + 1.0))
    ku_in = np.where(t <= s - 1.0, np.exp(-(s - t)), np.exp(-1.0))
    ku_pv = np.exp(-(s + BLK - t))
    kps = np.concatenate([kp_in, kp_nx], axis=0).astype(np.float32)
    kus = np.concatenate([ku_pv, ku_in], axis=0).astype(np.float32)
    return bld, kps, kus


_BLD_NP, _KPS_NP, _KUS_NP = _build_tables()
_BLD = jnp.asarray(_BLD_NP)
_KPS = jnp.asarray(_KPS_NP, dtype=jnp.bfloat16)
_KUS = jnp.asarray(_KUS_NP, dtype=jnp.bfloat16)


def _dp_kernel(ns_ref, scal_ref, dpad_ref, bld_ref, kps_ref, kus_ref, out_ref):
    lanes = jax.lax.broadcasted_iota(jnp.int32, (SUB, LANES), 1).astype(jnp.float32)
    scal = scal_ref[...]
    Mf = scal[:, 0:1]
    Nf = scal[:, 1:2]
    wsf = scal[:, 2:3]
    lds = scal[:, 3:4]
    ldo = scal[:, 4:5]

    one0 = jnp.where(lanes == 0.0, 1.0, 0.0)
    p0 = one0
    u0 = one0
    t0 = jnp.zeros((SUB, LANES), jnp.float32)
    z1 = jnp.zeros((SUB, 1), jnp.float32)
    o1 = jnp.ones((SUB, 1), jnp.float32)

    def body(i, carry):
        pc, uc, tacc, lkp, lku, spm, sum_ = carry
        fi = i.astype(jnp.float32)
        epb = pc.astype(jnp.bfloat16)
        eub = uc.astype(jnp.bfloat16)

        zblk = jnp.zeros((SUB, BLK), jnp.bfloat16)
        ep_blocks = [epb[:, b * BLK : (b + 1) * BLK] for b in range(NBLK)] + [zblk]
        eu_blocks = [zblk] + [eub[:, b * BLK : (b + 1) * BLK] for b in range(NBLK)]
        tp = [jnp.sum(pc[:, b * BLK : (b + 1) * BLK], axis=1, keepdims=True)
              for b in range(NBLK)]
        tu = [jnp.sum(uc[:, b * BLK : (b + 1) * BLK], axis=1, keepdims=True)
              for b in range(NBLK)]

        p_in = jnp.concatenate(
            [jnp.concatenate([ep_blocks[b], ep_blocks[b + 1]], axis=1)
             for b in range(NBLK)], axis=0)
        u_in = jnp.concatenate(
            [jnp.concatenate([eu_blocks[b], eu_blocks[b + 1]], axis=1)
             for b in range(NBLK)], axis=0)
        sp_all = jnp.dot(p_in, kps_ref[...], preferred_element_type=jnp.float32)
        su_all = jnp.dot(u_in, kus_ref[...], preferred_element_type=jnp.float32)

        einv = jnp.float32(0.36787944117144233)
        sp_blocks = []
        su_blocks = []
        cpre = z1
        csuf = z1
        for b in range(NBLK):
            sp_blocks.append(sp_all[b * SUB : (b + 1) * SUB] + cpre)
            cpre = cpre + tp[b]
        for b in range(NBLK - 1, -1, -1):
            su_blocks.append(su_all[b * SUB : (b + 1) * SUB] + einv * csuf)
            csuf = csuf + tu[b]
        sp = jnp.concatenate(sp_blocks, axis=1)
        su = jnp.concatenate(su_blocks[::-1], axis=1)

        dvalrow = dpad_ref[i] + bld_ref[i] * lds + ldo
        lo = jnp.maximum(1.0, fi - wsf)
        hi = jnp.minimum(Nf + 1.0, fi + wsf)
        bandt = (lanes >= lo) & (lanes < hi)
        dexp = jnp.where(bandt, jnp.exp(-dvalrow), 0.0)
        dexp_p = dexp * (1.0 / spm)
        dexp_u = dexp * (1.0 / sum_)

        tval = dvalrow + jnp.minimum(
            lkp - jnp.log(sp), lku - lanes - jnp.log(su)
        )
        hit = (fi == Mf) & (lanes == Nf)
        tacc = tacc + jnp.where(hit, tval, 0.0)

        lkp2 = lkp - jnp.log(spm)
        lku2 = lku - jnp.log(sum_)
        pn = dexp_p * sp
        un = dexp_u * su
        spm2 = jnp.max(pn, axis=1, keepdims=True)
        sum2 = jnp.max(un, axis=1, keepdims=True)
        return pn, un, tacc, lkp2, lku2, spm2, sum2

    def body8(idx, carry):
        for r in range(16):
            carry = body(16 * idx + 1 + r, carry)
        return carry

    carry = (p0, u0, t0, z1, z1, o1, o1)
    carry = jax.lax.fori_loop(0, ns_ref[0], body8, carry)
    tacc = carry[2]
    out_ref[...] = jnp.broadcast_to(
        jnp.sum(tacc, axis=1, keepdims=True), (SUB, 128)
    )


@jax.jit
def kernel(D, len_a, len_b):
    B, L1, L2 = D.shape

    Mf = len_a.astype(jnp.float32)
    Nf = len_b.astype(jnp.float32)
    dM = jnp.abs(Mf - Nf)
    wsf = jnp.maximum(5.0, dM + 1.0)
    lds = LBD_DUR / Mf
    ldo = dM * LBD_LEN / Mf
    cols = jnp.stack([Mf, Nf, wsf, lds, ldo], axis=1)
    scal = jnp.zeros((SUB, 128), jnp.float32)
    scal = scal.at[:B, :5].set(cols)
    scal = scal.at[B:, 0].set(256.0)
    scal = scal.at[B:, 1].set(256.0)
    scal = scal.at[B:, 2].set(5.0)
    scal = scal.at[B:, 3].set(LBD_DUR / 256.0)

    nsteps = (jnp.max(len_a).astype(jnp.int32) + 15) // 16

    Dr = jnp.concatenate([D, D[:, -1:, :]], axis=1)
    Dt = jnp.transpose(Dr, (1, 0, 2))
    dpad = jnp.pad(Dt, ((1, 0), (0, SUB - B), (1, LANES - L2 - 1)))

    out = pl.pallas_call(
        _dp_kernel,
        out_shape=jax.ShapeDtypeStruct((SUB, 128), jnp.float32),
        in_specs=[
            pl.BlockSpec(memory_space=pltpu.SMEM),
            pl.BlockSpec(memory_space=pltpu.VMEM),
            pl.BlockSpec(memory_space=pltpu.VMEM),
            pl.BlockSpec(memory_space=pltpu.VMEM),
            pl.BlockSpec(memory_space=pltpu.VMEM),
            pl.BlockSpec(memory_space=pltpu.VMEM),
        ],
    )(nsteps.reshape(1), scal, dpad, _BLD, _KPS, _KUS)
    return out[:B, 0]


# speedup vs baseline: 1.0605x; 1.0109x over previous
import jax
import jax.numpy as jnp
from jax.experimental import pallas as pl
from jax.experimental.pallas import tpu as pltpu

GAMMA = 1.0
LBD_ORD = 1.0
LBD_DUR = 5.0
LBD_LEN = 0.2
SIGMA = 1.0
MARGIN = 1.0

import numpy as np

SUB = 8
LANES = 640
BLK = 128
NBLK = LANES // BLK
NROWS = 514


def _build_tables():
    ii = np.arange(NROWS, dtype=np.float64)[:, None]
    jj = np.arange(LANES, dtype=np.float64)[None, :]
    bld = (1.0 - np.exp(-((ii - jj) ** 2) / (2.0 * SIGMA**2 * (jj + 1.0))))
    bld = bld.astype(np.float32).reshape(NROWS, 1, LANES)
    t = np.arange(BLK, dtype=np.float64)[:, None]
    s = np.arange(BLK, dtype=np.float64)[None, :]
    kp_in = np.where(t <= s - 1.0, 1.0, np.exp(-(t - s + 1.0)))
    kp_nx = np.exp(-(BLK + t - s + 1.0))
    ku_in = np.where(t <= s - 1.0, np.exp(-(s - t)), np.exp(-1.0))
    ku_pv = np.exp(-(s + BLK - t))
    kps = np.concatenate([kp_in, kp_nx], axis=0).astype(np.float32)
    kus = np.concatenate([ku_pv, ku_in], axis=0).astype(np.float32)
    return bld, kps, kus


_BLD_NP, _KPS_NP, _KUS_NP = _build_tables()
_BLD = jnp.asarray(_BLD_NP)
_KPS = jnp.asarray(_KPS_NP, dtype=jnp.bfloat16)
_KUS = jnp.asarray(_KUS_NP, dtype=jnp.bfloat16)


def _dp_kernel(ns_ref, scal_ref, dpad_ref, bld_ref, kps_ref, kus_ref, out_ref):
    lanes = jax.lax.broadcasted_iota(jnp.int32, (SUB, LANES), 1).astype(jnp.float32)
    scal = scal_ref[...]
    Mf = scal[:, 0:1]
    Nf = scal[:, 1:2]
    wsf = scal[:, 2:3]
    lds = scal[:, 3:4]
    ldo = scal[:, 4:5]

    one0 = jnp.where(lanes == 0.0, 1.0, 0.0)
    p0 = one0
    u0 = one0
    t0 = jnp.zeros((SUB, LANES), jnp.float32)
    z1 = jnp.zeros((SUB, 1), jnp.float32)
    o1 = jnp.ones((SUB, 1), jnp.float32)

    def body(i, carry):
        pc, uc, tacc, lkp, lku, spm, sum_ = carry
        fi = i.astype(jnp.float32)
        epb = pc.astype(jnp.bfloat16)
        eub = uc.astype(jnp.bfloat16)

        zblk = jnp.zeros((SUB, BLK), jnp.bfloat16)
        ep_blocks = [epb[:, b * BLK : (b + 1) * BLK] for b in range(NBLK)] + [zblk]
        eu_blocks = [zblk] + [eub[:, b * BLK : (b + 1) * BLK] for b in range(NBLK)]
        tp = [jnp.sum(pc[:, b * BLK : (b + 1) * BLK], axis=1, keepdims=True)
              for b in range(NBLK)]
        tu = [jnp.sum(uc[:, b * BLK : (b + 1) * BLK], axis=1, keepdims=True)
              for b in range(NBLK)]

        p_in = jnp.concatenate(
            [jnp.concatenate([ep_blocks[b], ep_blocks[b + 1]], axis=1)
             for b in range(NBLK)], axis=0)
        u_in = jnp.concatenate(
            [jnp.concatenate([eu_blocks[b], eu_blocks[b + 1]], axis=1)
             for b in range(NBLK)], axis=0)
        sp_all = jnp.dot(p_in, kps_ref[...], preferred_element_type=jnp.float32)
        su_all = jnp.dot(u_in, kus_ref[...], preferred_element_type=jnp.float32)

        einv = jnp.float32(0.36787944117144233)
        sp_blocks = []
        su_blocks = []
        cpre = z1
        csuf = z1
        for b in range(NBLK):
            sp_blocks.append(sp_all[b * SUB : (b + 1) * SUB] + cpre)
            cpre = cpre + tp[b]
        for b in range(NBLK - 1, -1, -1):
            su_blocks.append(su_all[b * SUB : (b + 1) * SUB] + einv * csuf)
            csuf = csuf + tu[b]
        sp = jnp.concatenate(sp_blocks, axis=1)
        su = jnp.concatenate(su_blocks[::-1], axis=1)

        dvalrow = dpad_ref[i] + bld_ref[i] * lds + ldo
        lo = jnp.maximum(1.0, fi - wsf)
        hi = jnp.minimum(Nf + 1.0, fi + wsf)
        bandt = (lanes >= lo) & (lanes < hi)
        dexp = jnp.where(bandt, jnp.exp(-dvalrow), 0.0)
        dexp_p = dexp * (1.0 / spm)
        dexp_u = dexp * (1.0 / sum_)

        tval = dvalrow + jnp.minimum(
            lkp - jnp.log(sp), lku - lanes - jnp.log(su)
        )
        hit = (fi == Mf) & (lanes == Nf)
        tacc = tacc + jnp.where(hit, tval, 0.0)

        lkp2 = lkp - jnp.log(spm)
        lku2 = lku - jnp.log(sum_)
        pn = dexp_p * sp
        un = dexp_u * su
        spm2 = jnp.max(pn, axis=1, keepdims=True)
        sum2 = jnp.max(un, axis=1, keepdims=True)
        return pn, un, tacc, lkp2, lku2, spm2, sum2

    def body8(idx, carry):
        for r in range(16):
            carry = body(16 * idx + 1 + r, carry)
        return carry

    carry = (p0, u0, t0, z1, z1, o1, o1)
    carry = jax.lax.fori_loop(0, ns_ref[0], body8, carry)
    tacc = carry[2]
    out_ref[...] = jnp.broadcast_to(
        jnp.sum(tacc, axis=1, keepdims=True), (SUB, 128)
    )


@jax.jit
def kernel(D, len_a, len_b):
    B, L1, L2 = D.shape

    Mf = len_a.astype(jnp.float32)
    Nf = len_b.astype(jnp.float32)
    dM = jnp.abs(Mf - Nf)
    wsf = jnp.maximum(5.0, dM + 1.0)
    lds = LBD_DUR / Mf
    ldo = dM * LBD_LEN / Mf
    cols = jnp.stack([Mf, Nf, wsf, lds, ldo], axis=1)
    scal = jnp.zeros((SUB, 128), jnp.float32)
    scal = scal.at[:B, :5].set(cols)
    scal = scal.at[B:, 0].set(256.0)
    scal = scal.at[B:, 1].set(256.0)
    scal = scal.at[B:, 2].set(5.0)
    scal = scal.at[B:, 3].set(LBD_DUR / 256.0)

    nsteps = (jnp.max(len_a).astype(jnp.int32) + 15) // 16

    Dt = jnp.transpose(D, (1, 0, 2))
    dpad = jnp.pad(Dt, ((1, 0), (0, SUB - B), (1, LANES - L2 - 1)))

    out = pl.pallas_call(
        _dp_kernel,
        out_shape=jax.ShapeDtypeStruct((SUB, 128), jnp.float32),
        in_specs=[
            pl.BlockSpec(memory_space=pltpu.SMEM),
            pl.BlockSpec(memory_space=pltpu.VMEM),
            pl.BlockSpec(memory_space=pltpu.VMEM),
            pl.BlockSpec(memory_space=pltpu.VMEM),
            pl.BlockSpec(memory_space=pltpu.VMEM),
            pl.BlockSpec(memory_space=pltpu.VMEM),
        ],
    )(nsteps.reshape(1), scal, dpad, _BLD, _KPS, _KUS)
    return out[:B, 0]
